# baseline (speedup 1.0000x reference)
import math

import jax
import jax.numpy as jnp
from jax import lax
from jax.experimental import pallas as pl
from jax.experimental.pallas import tpu as pltpu
from jax.experimental.pallas import tpu_sc as plsc

_HIDDEN = 64
_SCALE = math.sqrt(float(_HIDDEN))
_SEQ = 200
_BATCH = 4096
_NC, _NS = 2, 16
_NW = _NC * _NS
_BPW = _BATCH // _NW
_CR = 2
_G = _SEQ // _CR
_NB = 3
_L = 16
_CH = _HIDDEN // 8


def _emb_body(xt_hbm, table_hbm, out_hbm, idx_v, rows_v, tout_v,
              gs0, gs1, gs2, ss0, ss1, ss2):
    gsems = (gs0, gs1, gs2)
    ssems = (ss0, ss1, ss2)
    wid = lax.axis_index("s") * _NC + lax.axis_index("c")

    lane = lax.iota(jnp.int32, _L)
    zero16 = jnp.zeros((_L,), jnp.int32)
    ch_idx = [lane // 8 + 2 * k for k in range(_HIDDEN // _L)]
    cl_idx = [lane % 8 for _ in range(_HIDDEN // _L)]

    def fire_gather(g, b):
        pltpu.sync_copy(xt_hbm.at[wid, pl.ds(g * _CR, _CR)], idx_v.at[b])
        for j in range(_CR):
            pltpu.async_copy(table_hbm.at[idx_v.at[b, j]], rows_v.at[b, j],
                             gsems[b])

    def drain_gather(b):
        for j in range(_CR):
            pltpu.make_async_copy(table_hbm.at[idx_v.at[b, j]],
                                  rows_v.at[b, j], gsems[b]).wait()

    def transpose_scale(b):
        for j in range(_CR):
            @pl.loop(0, _BPW, unroll=4)
            def _(bl):
                bls = zero16 + bl
                for k in range(_HIDDEN // _L):
                    v = rows_v[b, j, bl, pl.ds(k * _L, _L)] * _SCALE
                    plsc.store_scatter(tout_v.at[b, j],
                                       [ch_idx[k], zero16, cl_idx[k], bls],
                                       v)

    def fire_scatter(g, b):
        for j in range(_CR):
            pltpu.async_copy(
                tout_v.at[b, j, pl.ds(0, _CH), pl.ds(0, 1), pl.ds(0, 8),
                          pl.ds(0, _BPW)],
                out_hbm.at[pl.ds((g * _CR + j) * _CH, _CH), pl.ds(wid, 1)],
                ssems[b])

    def drain_scatter(g, b):
        for j in range(_CR):
            pltpu.make_async_copy(
                tout_v.at[b, j, pl.ds(0, _CH), pl.ds(0, 1), pl.ds(0, 8),
                          pl.ds(0, _BPW)],
                out_hbm.at[pl.ds((g * _CR + j) * _CH, _CH), pl.ds(wid, 1)],
                ssems[b]).wait()

    fire_gather(0, 0)

    @pl.loop(0, _G + (-_G % _NB), step=_NB)
    def _(g0):
        for b in range(_NB):
            g = g0 + b
            nb = (b + 1) % _NB

            @pl.when(g < _G)
            def _():
                @pl.when(g + 1 < _G)
                def _():
                    @pl.when(g >= 2)
                    def _():
                        drain_scatter(g - 2, nb)
                    fire_gather(g + 1, nb)

                drain_gather(b)
                transpose_scale(b)
                fire_scatter(g, b)

    for g in range(_G - _NB, _G):
        drain_scatter(g, g % _NB)


@jax.jit
def kernel(x, table):
    xt = x.T.reshape(_SEQ, _NW, _BPW).transpose(1, 0, 2)
    mesh = plsc.VectorSubcoreMesh(core_axis_name="c", subcore_axis_name="s")
    out = pl.kernel(
        _emb_body,
        out_type=jax.ShapeDtypeStruct((_SEQ * _CH, _NW, 8, _BPW),
                                      jnp.float32),
        mesh=mesh,
        compiler_params=pltpu.CompilerParams(use_tc_tiling_on_sc=False,
                                             needs_layout_passes=False),
        scratch_types=[
            pltpu.VMEM((_NB, _CR, _BPW), jnp.int32),
            pltpu.VMEM((_NB, _CR, _BPW, _HIDDEN), jnp.float32),
            pltpu.VMEM((_NB, _CR, _CH, 1, 8, _BPW + 9), jnp.float32),
            pltpu.SemaphoreType.DMA,
            pltpu.SemaphoreType.DMA,
            pltpu.SemaphoreType.DMA,
            pltpu.SemaphoreType.DMA,
            pltpu.SemaphoreType.DMA,
            pltpu.SemaphoreType.DMA,
        ],
    )(xt, table)
    out5 = out.reshape(_SEQ, _CH, _NW, 8, _BPW)
    return out5.transpose(2, 4, 0, 1, 3).reshape(_BATCH, _SEQ, _HIDDEN)


# speedup vs baseline: 1.0628x; 1.0628x over previous
import math

import jax
import jax.numpy as jnp
from jax import lax
from jax.experimental import pallas as pl
from jax.experimental.pallas import tpu as pltpu
from jax.experimental.pallas import tpu_sc as plsc

_HIDDEN = 64
_SCALE = math.sqrt(float(_HIDDEN))
_SEQ = 200
_BATCH = 4096
_NC, _NS = 2, 16
_NW = _NC * _NS
_BPW = _BATCH // _NW
_CR = 2
_G = _SEQ // _CR
_NB = 3
_L = 16
_CH = _HIDDEN // 8


def _emb_body(xt_hbm, table_hbm, out_hbm, idx_v, rows_v, tout_v,
              gs0, gs1, gs2, ss0, ss1, ss2):
    gsems = (gs0, gs1, gs2)
    ssems = (ss0, ss1, ss2)
    wid = lax.axis_index("s") * _NC + lax.axis_index("c")

    lane = lax.iota(jnp.int32, _L)
    zero16 = jnp.zeros((_L,), jnp.int32)
    ch_idx = [lane // 8 + 2 * k for k in range(_HIDDEN // _L)]
    cl_idx = [lane % 8 for _ in range(_HIDDEN // _L)]

    def fire_gather(g, b):
        pltpu.sync_copy(xt_hbm.at[wid, pl.ds(g * _CR, _CR)], idx_v.at[b])
        for j in range(_CR):
            for h in range(4):
                pltpu.async_copy(
                    table_hbm.at[idx_v.at[b, j, pl.ds(h * 32, 32)]],
                    rows_v.at[b, j, pl.ds(h * 32, 32)], gsems[b])

    def drain_gather(b):
        for j in range(_CR):
            for h in range(4):
                pltpu.make_async_copy(
                    table_hbm.at[idx_v.at[b, j, pl.ds(h * 32, 32)]],
                    rows_v.at[b, j, pl.ds(h * 32, 32)], gsems[b]).wait()

    def transpose_scale(b):
        for j in range(_CR):
            @pl.loop(0, _BPW, unroll=4)
            def _(bl):
                bls = zero16 + bl
                for k in range(_HIDDEN // _L):
                    v = rows_v[b, j, bl, pl.ds(k * _L, _L)] * _SCALE
                    plsc.store_scatter(tout_v.at[b, j],
                                       [ch_idx[k], zero16, cl_idx[k], bls],
                                       v)

    def fire_scatter(g, b):
        for j in range(_CR):
            pltpu.async_copy(
                tout_v.at[b, j, pl.ds(0, _CH), pl.ds(0, 1), pl.ds(0, 8),
                          pl.ds(0, _BPW)],
                out_hbm.at[pl.ds((g * _CR + j) * _CH, _CH), pl.ds(wid, 1)],
                ssems[b])

    def drain_scatter(g, b):
        for j in range(_CR):
            pltpu.make_async_copy(
                tout_v.at[b, j, pl.ds(0, _CH), pl.ds(0, 1), pl.ds(0, 8),
                          pl.ds(0, _BPW)],
                out_hbm.at[pl.ds((g * _CR + j) * _CH, _CH), pl.ds(wid, 1)],
                ssems[b]).wait()

    fire_gather(0, 0)

    @pl.loop(0, _G + (-_G % _NB), step=_NB)
    def _(g0):
        for b in range(_NB):
            g = g0 + b
            nb = (b + 1) % _NB

            @pl.when(g < _G)
            def _():
                @pl.when(g + 1 < _G)
                def _():
                    @pl.when(g >= 2)
                    def _():
                        drain_scatter(g - 2, nb)
                    fire_gather(g + 1, nb)

                drain_gather(b)
                transpose_scale(b)
                fire_scatter(g, b)

    for g in range(_G - _NB, _G):
        drain_scatter(g, g % _NB)


@jax.jit
def kernel(x, table):
    xt = x.T.reshape(_SEQ, _NW, _BPW).transpose(1, 0, 2)
    mesh = plsc.VectorSubcoreMesh(core_axis_name="c", subcore_axis_name="s")
    out = pl.kernel(
        _emb_body,
        out_type=jax.ShapeDtypeStruct((_SEQ * _CH, _NW, 8, _BPW),
                                      jnp.float32),
        mesh=mesh,
        compiler_params=pltpu.CompilerParams(use_tc_tiling_on_sc=False,
                                             needs_layout_passes=False),
        scratch_types=[
            pltpu.VMEM((_NB, _CR, _BPW), jnp.int32),
            pltpu.VMEM((_NB, _CR, _BPW, _HIDDEN), jnp.float32),
            pltpu.VMEM((_NB, _CR, _CH, 1, 8, _BPW + 1), jnp.float32),
            pltpu.SemaphoreType.DMA,
            pltpu.SemaphoreType.DMA,
            pltpu.SemaphoreType.DMA,
            pltpu.SemaphoreType.DMA,
            pltpu.SemaphoreType.DMA,
            pltpu.SemaphoreType.DMA,
        ],
    )(xt, table)
    out5 = out.reshape(_SEQ, _CH, _NW, 8, _BPW)
    return out5.transpose(2, 4, 0, 1, 3).reshape(_BATCH, _SEQ, _HIDDEN)


# speedup vs baseline: 1.0659x; 1.0029x over previous
import math

import jax
import jax.numpy as jnp
from jax import lax
from jax.experimental import pallas as pl
from jax.experimental.pallas import tpu as pltpu
from jax.experimental.pallas import tpu_sc as plsc

_HIDDEN = 64
_SCALE = math.sqrt(float(_HIDDEN))
_SEQ = 200
_BATCH = 4096
_NC, _NS = 2, 16
_NW = _NC * _NS
_BPW = _BATCH // _NW
_CR = 2
_G = _SEQ // _CR
_NB = 3
_L = 16
_CH = _HIDDEN // 8


def _emb_body(xt_hbm, table_hbm, out_hbm, idx_v, rows_v, tout_v,
              gs0, gs1, gs2, ss0, ss1, ss2):
    gsems = (gs0, gs1, gs2)
    ssems = (ss0, ss1, ss2)
    wid = lax.axis_index("s") * _NC + lax.axis_index("c")

    lane = lax.iota(jnp.int32, _L)
    zero16 = jnp.zeros((_L,), jnp.int32)
    ch_idx = [lane // 8 + 2 * k for k in range(_HIDDEN // _L)]
    cl_idx = [lane % 8 for _ in range(_HIDDEN // _L)]

    def fire_gather(g, b):
        pltpu.sync_copy(xt_hbm.at[wid, pl.ds(g * _CR, _CR)], idx_v.at[b])
        for j in range(_CR):
            pltpu.async_copy(table_hbm.at[idx_v.at[b, j]], rows_v.at[b, j],
                             gsems[b])

    def drain_gather(b):
        for j in range(_CR):
            pltpu.make_async_copy(table_hbm.at[idx_v.at[b, j]],
                                  rows_v.at[b, j], gsems[b]).wait()

    def transpose_scale(b):
        for j in range(_CR):
            @pl.loop(0, _BPW, unroll=4)
            def _(bl):
                bls = zero16 + bl
                for k in range(_HIDDEN // _L):
                    v = rows_v[b, j, bl, pl.ds(k * _L, _L)] * _SCALE
                    plsc.store_scatter(tout_v.at[b, j],
                                       [ch_idx[k], zero16, cl_idx[k], bls],
                                       v)

    def fire_scatter(g, b):
        for j in range(_CR):
            pltpu.async_copy(
                tout_v.at[b, j, pl.ds(0, _CH), pl.ds(0, 1), pl.ds(0, 8),
                          pl.ds(0, _BPW)],
                out_hbm.at[pl.ds((g * _CR + j) * _CH, _CH), pl.ds(wid, 1)],
                ssems[b])

    def drain_scatter(g, b):
        for j in range(_CR):
            pltpu.make_async_copy(
                tout_v.at[b, j, pl.ds(0, _CH), pl.ds(0, 1), pl.ds(0, 8),
                          pl.ds(0, _BPW)],
                out_hbm.at[pl.ds((g * _CR + j) * _CH, _CH), pl.ds(wid, 1)],
                ssems[b]).wait()

    fire_gather(0, 0)

    @pl.loop(0, _G + (-_G % _NB), step=_NB)
    def _(g0):
        for b in range(_NB):
            g = g0 + b
            nb = (b + 1) % _NB

            @pl.when(g < _G)
            def _():
                @pl.when(g + 1 < _G)
                def _():
                    @pl.when(g >= 2)
                    def _():
                        drain_scatter(g - 2, nb)
                    fire_gather(g + 1, nb)

                drain_gather(b)
                transpose_scale(b)
                fire_scatter(g, b)

    for g in range(_G - _NB, _G):
        drain_scatter(g, g % _NB)


@jax.jit
def kernel(x, table):
    xt = x.T.reshape(_SEQ, _NW, _BPW).transpose(1, 0, 2)
    mesh = plsc.VectorSubcoreMesh(core_axis_name="c", subcore_axis_name="s")
    out = pl.kernel(
        _emb_body,
        out_type=jax.ShapeDtypeStruct((_SEQ * _CH, _NW, 8, _BPW),
                                      jnp.float32),
        mesh=mesh,
        compiler_params=pltpu.CompilerParams(use_tc_tiling_on_sc=False,
                                             needs_layout_passes=False),
        scratch_types=[
            pltpu.VMEM((_NB, _CR, _BPW), jnp.int32),
            pltpu.VMEM((_NB, _CR, _BPW, _HIDDEN), jnp.float32),
            pltpu.VMEM((_NB, _CR, _CH, 1, 8, _BPW + 1), jnp.float32),
            pltpu.SemaphoreType.DMA,
            pltpu.SemaphoreType.DMA,
            pltpu.SemaphoreType.DMA,
            pltpu.SemaphoreType.DMA,
            pltpu.SemaphoreType.DMA,
            pltpu.SemaphoreType.DMA,
        ],
    )(xt, table)
    out5 = out.reshape(_SEQ, _CH, _NW, 8, _BPW)
    return out5.transpose(2, 4, 0, 1, 3).reshape(_BATCH, _SEQ, _HIDDEN)
